# baseline (speedup 1.0000x reference)
import functools

import jax
import jax.numpy as jnp
from jax import lax
from jax.experimental import pallas as pl
from jax.experimental.pallas import tpu as pltpu
from jax.experimental.pallas import tpu_sc as plsc

N_NEIGHBORS = 20
THRESHOLD = -0.5
BIG = 3.0e38


def _enc_body(x_ref, wenc_ref, wcls_ref, emb_ref, preds_ref):
    x = x_ref[...]
    emb = jnp.dot(x, wenc_ref[...], preferred_element_type=jnp.float32)
    emb_ref[...] = emb
    logits = jnp.dot(emb, wcls_ref[...], preferred_element_type=jnp.float32)
    am = jnp.argmax(logits, axis=1).astype(jnp.int32)
    preds_ref[...] = am[:, None]


def _knn_body(emb_ref, keys_ref, ld_ref, idx_ref, topv_ref, topi_ref,
              *, n_chunks, chunk, k_valid):
    i = pl.program_id(0)
    q = emb_ref.shape[0]

    @pl.when(i == 0)
    def _init():
        lane = lax.broadcasted_iota(jnp.int32, (q, 128), 1)
        topv_ref[...] = jnp.where(lane < N_NEIGHBORS,
                                  jnp.full((q, 128), BIG, jnp.float32),
                                  jnp.full((q, 128), -1.0, jnp.float32))
        topi_ref[...] = jnp.zeros((q, 128), jnp.int32)

    emb = emb_ref[...]
    keys = keys_ref[...]
    mm = lax.dot_general(emb, keys, (((1,), (1,)), ((), ())),
                         preferred_element_type=jnp.float32)
    q2 = jnp.sum(emb * emb, axis=1, keepdims=True)
    k2 = jnp.sum(keys * keys, axis=1)
    d2 = q2 - 2.0 * mm + k2[None, :]
    dist = jnp.sqrt(jnp.maximum(d2, 0.0) + 1e-12)
    col = lax.broadcasted_iota(jnp.int32, (q, chunk), 1)
    gcol = i * chunk + col
    dist = jnp.where(gcol < k_valid, dist, BIG)

    lane = lax.broadcasted_iota(jnp.int32, (q, 128), 1)

    topv0 = topv_ref[...]
    topi0 = topi_ref[...]
    worst0 = jnp.max(topv0, axis=1, keepdims=True)
    cnt = jnp.sum((dist < worst0).astype(jnp.int32), axis=1, keepdims=True)
    nmax = jnp.minimum(jnp.max(cnt), N_NEIGHBORS)

    def body(_, carry):
        dist, topv, topi = carry
        m = jnp.min(dist, axis=1, keepdims=True)
        am = jnp.argmin(dist, axis=1).astype(jnp.int32)
        gi = i * chunk + am
        worst = jnp.max(topv, axis=1, keepdims=True)
        aw = jnp.argmax(topv, axis=1).astype(jnp.int32)
        upd = (lane == aw[:, None]) & (m < worst)
        topv = jnp.where(upd, m, topv)
        topi = jnp.where(upd, gi[:, None], topi)
        dist = jnp.where(col == am[:, None], BIG, dist)
        return dist, topv, topi

    _, topv, topi = lax.fori_loop(0, nmax, body, (dist, topv0, topi0))
    topv_ref[...] = topv
    topi_ref[...] = topi

    @pl.when(i == n_chunks - 1)
    def _fin():
        tv = jnp.where(lane < N_NEIGHBORS, topv_ref[...], 0.0)
        meanknn = jnp.sum(tv, axis=1, keepdims=True) / N_NEIGHBORS
        ld_ref[...] = 1.0 / (meanknn + 1e-10)
        idx_ref[...] = topi_ref[...]


def _make_sc_inflo(kn, qn, n_tiles, qpt):
    mesh = plsc.VectorSubcoreMesh(core_axis_name="c", subcore_axis_name="s")
    groups = qpt // 16

    @functools.partial(
        pl.kernel,
        mesh=mesh,
        out_type=[
            jax.ShapeDtypeStruct((qn,), jnp.float32),
            jax.ShapeDtypeStruct((qn,), jnp.int32),
            jax.ShapeDtypeStruct((qn,), jnp.int32),
        ],
        scratch_types=[
            pltpu.VMEM((kn,), jnp.float32),
            pltpu.VMEM((N_NEIGHBORS, qpt), jnp.int32),
            pltpu.VMEM((qpt,), jnp.float32),
            pltpu.VMEM((qpt,), jnp.int32),
            pltpu.VMEM((qpt,), jnp.float32),
            pltpu.VMEM((qpt,), jnp.int32),
            pltpu.VMEM((qpt,), jnp.int32),
        ],
        compiler_params=pltpu.CompilerParams(needs_layout_passes=False),
    )
    def sc_body(kmkd_hbm, idx_hbm, ld_hbm, pr_hbm,
                inflo_hbm, fl_hbm, po_hbm,
                table_v, idx_v, ld_v, pr_v, inflo_v, fl_v, po_v):
        wid = lax.axis_index("s") * 2 + lax.axis_index("c")
        base = wid * qpt
        pltpu.sync_copy(kmkd_hbm, table_v)
        pltpu.sync_copy(idx_hbm.at[wid], idx_v)
        pltpu.sync_copy(ld_hbm.at[pl.ds(base, qpt)], ld_v)
        pltpu.sync_copy(pr_hbm.at[pl.ds(base, qpt)], pr_v)
        for g in range(groups):
            sl = pl.ds(g * 16, 16)
            acc = jnp.zeros((16,), jnp.float32)
            eps = jnp.full((16,), 1e-10, jnp.float32)
            for j in range(N_NEIGHBORS):
                iv = idx_v[j, sl]
                vals = plsc.load_gather(table_v, [iv])
                acc = acc + jnp.full((16,), 1.0, jnp.float32) / (vals + eps)
            avg = acc * jnp.full((16,), 1.0 / N_NEIGHBORS, jnp.float32)
            inflo = -(ld_v[sl] / (avg + eps))
            flag = inflo < jnp.full((16,), THRESHOLD, jnp.float32)
            inflo_v[sl] = inflo
            fl_v[sl] = jnp.where(flag, jnp.full((16,), 1, jnp.int32),
                                 jnp.full((16,), 0, jnp.int32))
            po_v[sl] = jnp.where(flag, jnp.full((16,), -1, jnp.int32),
                                 pr_v[sl])
        pltpu.sync_copy(inflo_v, inflo_hbm.at[pl.ds(base, qpt)])
        pltpu.sync_copy(fl_v, fl_hbm.at[pl.ds(base, qpt)])
        pltpu.sync_copy(po_v, po_hbm.at[pl.ds(base, qpt)])

    return sc_body


def kernel(x, keys, key_mean_knn_dist, W_enc, W_cls):
    qn, _ = x.shape
    kn, d = keys.shape
    chunk = 4096
    n_chunks = (kn + chunk - 1) // chunk
    kpad = n_chunks * chunk
    keys_p = jnp.pad(keys, ((0, kpad - kn), (0, 0)))

    emb, preds = pl.pallas_call(
        _enc_body,
        out_shape=(
            jax.ShapeDtypeStruct((qn, d), jnp.float32),
            jax.ShapeDtypeStruct((qn, 1), jnp.int32),
        ),
    )(x, W_enc, W_cls)

    ld, idx = pl.pallas_call(
        functools.partial(_knn_body, n_chunks=n_chunks, chunk=chunk,
                          k_valid=kn),
        grid=(n_chunks,),
        in_specs=[
            pl.BlockSpec((qn, d), lambda i: (0, 0)),
            pl.BlockSpec((chunk, d), lambda i: (i, 0)),
        ],
        out_specs=(
            pl.BlockSpec((qn, 1), lambda i: (0, 0)),
            pl.BlockSpec((qn, 128), lambda i: (0, 0)),
        ),
        out_shape=(
            jax.ShapeDtypeStruct((qn, 1), jnp.float32),
            jax.ShapeDtypeStruct((qn, 128), jnp.int32),
        ),
        scratch_shapes=[
            pltpu.VMEM((qn, 128), jnp.float32),
            pltpu.VMEM((qn, 128), jnp.int32),
        ],
        compiler_params=pltpu.CompilerParams(
            dimension_semantics=("arbitrary",)),
    )(emb, keys_p)

    n_tiles = 32
    qpt = qn // n_tiles
    idx20 = idx[:, :N_NEIGHBORS]
    idx_t = idx20.reshape(n_tiles, qpt, N_NEIGHBORS).transpose(0, 2, 1)
    sc = _make_sc_inflo(kn, qn, n_tiles, qpt)
    inflo_scores, fl, cls_preds = sc(key_mean_knn_dist, idx_t,
                                     ld[:, 0], preds[:, 0])
    return fl.astype(jnp.bool_), cls_preds, inflo_scores


# speedup vs baseline: 1.4665x; 1.4665x over previous
import functools

import jax
import jax.numpy as jnp
from jax import lax
from jax.experimental import pallas as pl
from jax.experimental.pallas import tpu as pltpu
from jax.experimental.pallas import tpu_sc as plsc

N_NEIGHBORS = 20
THRESHOLD = -0.5
BIG = 3.0e38


def _enc_body(x_ref, wenc_ref, wcls_ref, emb_ref, preds_ref):
    x = x_ref[...]
    emb = jnp.dot(x, wenc_ref[...], preferred_element_type=jnp.float32)
    emb_ref[...] = emb
    logits = jnp.dot(emb, wcls_ref[...], preferred_element_type=jnp.float32)
    am = jnp.argmax(logits, axis=1).astype(jnp.int32)
    preds_ref[...] = am[:, None]


def _knn_body(emb_ref, keys_ref, ld_ref, idx_ref, topv_ref, topi_ref,
              *, n_chunks, chunk, k_valid):
    i = pl.program_id(0)
    q = emb_ref.shape[0]

    @pl.when(i == 0)
    def _init():
        lane = lax.broadcasted_iota(jnp.int32, (q, 128), 1)
        topv_ref[...] = jnp.where(lane < N_NEIGHBORS,
                                  jnp.full((q, 128), BIG, jnp.float32),
                                  jnp.full((q, 128), -1.0, jnp.float32))
        topi_ref[...] = jnp.zeros((q, 128), jnp.int32)

    emb = emb_ref[...]
    keys = keys_ref[...]
    mm = lax.dot_general(emb, keys, (((1,), (1,)), ((), ())),
                         preferred_element_type=jnp.float32)
    q2 = jnp.sum(emb * emb, axis=1, keepdims=True)
    k2 = jnp.sum(keys * keys, axis=1)
    d2 = q2 - 2.0 * mm + k2[None, :]
    dist = jnp.sqrt(jnp.maximum(d2, 0.0) + 1e-12)
    col = lax.broadcasted_iota(jnp.int32, (q, chunk), 1)
    gcol = i * chunk + col
    dist = jnp.where(gcol < k_valid, dist, BIG)

    lane = lax.broadcasted_iota(jnp.int32, (q, 128), 1)

    topv0 = topv_ref[...]
    topi0 = topi_ref[...]
    worst0 = jnp.max(topv0, axis=1, keepdims=True)
    cnt = jnp.sum((dist < worst0).astype(jnp.int32), axis=1, keepdims=True)
    nmax = jnp.minimum(jnp.max(cnt), N_NEIGHBORS)

    def body(_, carry):
        dist, topv, topi = carry
        m = jnp.min(dist, axis=1, keepdims=True)
        am = jnp.argmin(dist, axis=1).astype(jnp.int32)
        gi = i * chunk + am
        worst = jnp.max(topv, axis=1, keepdims=True)
        aw = jnp.argmax(topv, axis=1).astype(jnp.int32)
        upd = (lane == aw[:, None]) & (m < worst)
        topv = jnp.where(upd, m, topv)
        topi = jnp.where(upd, gi[:, None], topi)
        dist = jnp.where(col == am[:, None], BIG, dist)
        return dist, topv, topi

    _, topv, topi = lax.fori_loop(0, nmax, body, (dist, topv0, topi0))
    topv_ref[...] = topv
    topi_ref[...] = topi

    @pl.when(i == n_chunks - 1)
    def _fin():
        tv = jnp.where(lane < N_NEIGHBORS, topv_ref[...], 0.0)
        meanknn = jnp.sum(tv, axis=1, keepdims=True) / N_NEIGHBORS
        ld_ref[...] = 1.0 / (meanknn + 1e-10)
        idx_ref[...] = topi_ref[...]


def _make_sc_inflo(kn, qn, n_tiles, qpt):
    mesh = plsc.VectorSubcoreMesh(core_axis_name="c", subcore_axis_name="s")
    groups = qpt // 16

    @functools.partial(
        pl.kernel,
        mesh=mesh,
        out_type=[
            jax.ShapeDtypeStruct((qn,), jnp.float32),
            jax.ShapeDtypeStruct((qn,), jnp.int32),
            jax.ShapeDtypeStruct((qn,), jnp.int32),
        ],
        scratch_types=[
            pltpu.VMEM((kn,), jnp.float32),
            pltpu.VMEM((N_NEIGHBORS, qpt), jnp.int32),
            pltpu.VMEM((qpt,), jnp.float32),
            pltpu.VMEM((qpt,), jnp.int32),
            pltpu.VMEM((qpt,), jnp.float32),
            pltpu.VMEM((qpt,), jnp.int32),
            pltpu.VMEM((qpt,), jnp.int32),
        ],
        compiler_params=pltpu.CompilerParams(needs_layout_passes=False),
    )
    def sc_body(kmkd_hbm, idx_hbm, ld_hbm, pr_hbm,
                inflo_hbm, fl_hbm, po_hbm,
                table_v, idx_v, ld_v, pr_v, inflo_v, fl_v, po_v):
        wid = lax.axis_index("s") * 2 + lax.axis_index("c")
        base = wid * qpt
        pltpu.sync_copy(kmkd_hbm, table_v)
        pltpu.sync_copy(idx_hbm.at[wid], idx_v)
        pltpu.sync_copy(ld_hbm.at[pl.ds(base, qpt)], ld_v)
        pltpu.sync_copy(pr_hbm.at[pl.ds(base, qpt)], pr_v)
        for g in range(groups):
            sl = pl.ds(g * 16, 16)
            acc = jnp.zeros((16,), jnp.float32)
            eps = jnp.full((16,), 1e-10, jnp.float32)
            for j in range(N_NEIGHBORS):
                iv = idx_v[j, sl]
                vals = plsc.load_gather(table_v, [iv])
                acc = acc + jnp.full((16,), 1.0, jnp.float32) / (vals + eps)
            avg = acc * jnp.full((16,), 1.0 / N_NEIGHBORS, jnp.float32)
            inflo = -(ld_v[sl] / (avg + eps))
            flag = inflo < jnp.full((16,), THRESHOLD, jnp.float32)
            inflo_v[sl] = inflo
            fl_v[sl] = jnp.where(flag, jnp.full((16,), 1, jnp.int32),
                                 jnp.full((16,), 0, jnp.int32))
            po_v[sl] = jnp.where(flag, jnp.full((16,), -1, jnp.int32),
                                 pr_v[sl])
        pltpu.sync_copy(inflo_v, inflo_hbm.at[pl.ds(base, qpt)])
        pltpu.sync_copy(fl_v, fl_hbm.at[pl.ds(base, qpt)])
        pltpu.sync_copy(po_v, po_hbm.at[pl.ds(base, qpt)])

    return sc_body


def kernel(x, keys, key_mean_knn_dist, W_enc, W_cls):
    qn, _ = x.shape
    kn, d = keys.shape
    chunk = 1024
    n_chunks = (kn + chunk - 1) // chunk
    kpad = n_chunks * chunk
    keys_p = jnp.pad(keys, ((0, kpad - kn), (0, 0)))

    emb, preds = pl.pallas_call(
        _enc_body,
        out_shape=(
            jax.ShapeDtypeStruct((qn, d), jnp.float32),
            jax.ShapeDtypeStruct((qn, 1), jnp.int32),
        ),
    )(x, W_enc, W_cls)

    ld, idx = pl.pallas_call(
        functools.partial(_knn_body, n_chunks=n_chunks, chunk=chunk,
                          k_valid=kn),
        grid=(n_chunks,),
        in_specs=[
            pl.BlockSpec((qn, d), lambda i: (0, 0)),
            pl.BlockSpec((chunk, d), lambda i: (i, 0)),
        ],
        out_specs=(
            pl.BlockSpec((qn, 1), lambda i: (0, 0)),
            pl.BlockSpec((qn, 128), lambda i: (0, 0)),
        ),
        out_shape=(
            jax.ShapeDtypeStruct((qn, 1), jnp.float32),
            jax.ShapeDtypeStruct((qn, 128), jnp.int32),
        ),
        scratch_shapes=[
            pltpu.VMEM((qn, 128), jnp.float32),
            pltpu.VMEM((qn, 128), jnp.int32),
        ],
        compiler_params=pltpu.CompilerParams(
            dimension_semantics=("arbitrary",)),
    )(emb, keys_p)

    n_tiles = 32
    qpt = qn // n_tiles
    idx20 = idx[:, :N_NEIGHBORS]
    idx_t = idx20.reshape(n_tiles, qpt, N_NEIGHBORS).transpose(0, 2, 1)
    sc = _make_sc_inflo(kn, qn, n_tiles, qpt)
    inflo_scores, fl, cls_preds = sc(key_mean_knn_dist, idx_t,
                                     ld[:, 0], preds[:, 0])
    return fl.astype(jnp.bool_), cls_preds, inflo_scores
